# baseline (speedup 1.0000x reference)
import functools

import jax
import jax.numpy as jnp
from jax import lax
from jax.experimental import pallas as pl
from jax.experimental.pallas import tpu as pltpu
from jax.experimental.pallas import tpu_sc as plsc

N = 10000
E = 320000
D = 128

NC = 2
NS = 16
NW = NC * NS
CHUNK = 128
NCH = 80
KF = 8
EP = NW * NCH * CHUNK
NP = 10240
ROWS_PER_TILE = NP // NS


def _sc_segment_sum(feat, srcp, dstp):
    zf = jnp.zeros((ROWS_PER_TILE, D), jnp.float32)
    ones = jnp.ones((CHUNK, D), jnp.float32)

    mesh = plsc.VectorSubcoreMesh(core_axis_name="c", subcore_axis_name="s",
                                  num_cores=NC, num_subcores=NS)

    @functools.partial(
        pl.kernel,
        out_type=(
            jax.ShapeDtypeStruct((NC, NP, D), jnp.float32),
            jax.ShapeDtypeStruct((NC, NP, D), jnp.float32),
        ),
        mesh=mesh,
        scratch_types=[
            pltpu.VMEM((CHUNK,), jnp.int32),
            pltpu.VMEM((CHUNK,), jnp.int32),
            pltpu.VMEM((KF, CHUNK), jnp.int32),
            pltpu.VMEM((CHUNK, D), jnp.float32),
            pltpu.VMEM((CHUNK, D), jnp.float32),
            pltpu.VMEM_SHARED((NP, D), jnp.float32),
            pltpu.SemaphoreType.DMA,
        ],
    )
    def sc_kernel(feat_hbm, src_hbm, dst_hbm, zf_hbm, ones_hbm,
                  sum_hbm, deg_hbm,
                  src_v, dst_v, dst8_v, rows_v, ones_v, acc_s, sem_a):
        c = lax.axis_index("c")
        s = lax.axis_index("s")
        wid = s * NC + c
        base = s * ROWS_PER_TILE

        pltpu.sync_copy(zf_hbm, acc_s.at[pl.ds(base, ROWS_PER_TILE)])
        pltpu.sync_copy(ones_hbm, ones_v)
        plsc.subcore_barrier()

        def feat_body(j, carry):
            pltpu.sync_copy(src_hbm.at[wid, j], src_v)
            pltpu.sync_copy(dst_hbm.at[wid, j], dst_v)
            pltpu.async_copy(feat_hbm.at[src_v], rows_v, sem_a).wait()
            pltpu.sync_copy(rows_v, acc_s.at[dst_v], add=True)
            return carry

        lax.fori_loop(0, NCH, feat_body, 0)
        plsc.subcore_barrier()

        pltpu.sync_copy(acc_s.at[pl.ds(base, ROWS_PER_TILE)],
                        sum_hbm.at[c, pl.ds(base, ROWS_PER_TILE), :])
        pltpu.sync_copy(zf_hbm, acc_s.at[pl.ds(base, ROWS_PER_TILE)])
        plsc.subcore_barrier()

        def deg_body(t, carry):
            j0 = t * KF
            pltpu.sync_copy(dst_hbm.at[wid, pl.ds(j0, KF)], dst8_v)
            for k in range(KF):
                pltpu.async_copy(ones_v, acc_s.at[dst8_v.at[k]], sem_a,
                                 add=True)
            for k in range(KF):
                pltpu.make_async_copy(ones_v, acc_s.at[dst8_v.at[k]],
                                      sem_a).wait()
            return carry

        lax.fori_loop(0, NCH // KF, deg_body, 0)
        plsc.subcore_barrier()

        pltpu.sync_copy(acc_s.at[pl.ds(base, ROWS_PER_TILE)],
                        deg_hbm.at[c, pl.ds(base, ROWS_PER_TILE), :])

    return sc_kernel(feat, srcp, dstp, zf, ones)


def _tc_body(sum_ref, deg_ref, feat_ref, wn_ref, ws_ref, b_ref, out_ref):
    s = sum_ref[0] + sum_ref[1]
    d = deg_ref[0, :, 0:1] + deg_ref[1, :, 0:1]
    mean = s / jnp.maximum(d, 1.0)
    h = jnp.dot(mean, wn_ref[...], preferred_element_type=jnp.float32)
    h += jnp.dot(feat_ref[...], ws_ref[...], preferred_element_type=jnp.float32)
    out_ref[...] = h + b_ref[...]


def _tc_combine(sums, degs, featp, W_neigh, W_self, b_self):
    blk = 1024
    grid = NP // blk
    return pl.pallas_call(
        _tc_body,
        grid=(grid,),
        in_specs=[
            pl.BlockSpec((NC, blk, D), lambda i: (0, i, 0)),
            pl.BlockSpec((NC, blk, D), lambda i: (0, i, 0)),
            pl.BlockSpec((blk, D), lambda i: (i, 0)),
            pl.BlockSpec((D, D), lambda i: (0, 0)),
            pl.BlockSpec((D, D), lambda i: (0, 0)),
            pl.BlockSpec((1, D), lambda i: (0, 0)),
        ],
        out_specs=pl.BlockSpec((blk, D), lambda i: (i, 0)),
        out_shape=jax.ShapeDtypeStruct((NP, D), jnp.float32),
    )(sums, degs, featp, W_neigh, W_self, b_self)


@jax.jit
def kernel(feat, edge_index, W_neigh, W_self, b_self):
    src = edge_index[0].astype(jnp.int32)
    dst = edge_index[1].astype(jnp.int32)
    pad = EP - E
    srcp = jnp.concatenate([src, jnp.zeros((pad,), jnp.int32)]).reshape(
        NW, NCH, CHUNK)
    dummy = N + (jnp.arange(pad, dtype=jnp.int32) % (NP - N))
    dstp = jnp.concatenate([dst, dummy]).reshape(NW, NCH, CHUNK)

    sums, degs = _sc_segment_sum(feat, srcp, dstp)

    featp = jnp.concatenate(
        [feat, jnp.zeros((NP - N, D), jnp.float32)], axis=0)
    out = _tc_combine(sums, degs, featp, W_neigh, W_self,
                      b_self.reshape(1, D))
    return out[:N]


# speedup vs baseline: 1.4169x; 1.4169x over previous
import functools

import jax
import jax.numpy as jnp
from jax import lax
from jax.experimental import pallas as pl
from jax.experimental.pallas import tpu as pltpu
from jax.experimental.pallas import tpu_sc as plsc

N = 10000
E = 320000
D = 128

NC = 2
NS = 16
NW = NC * NS
CHUNK = 128
NCH = 79
EP = NW * NCH * CHUNK
NP = 10240
ROWS_PER_TILE = NP // NS


def _sc_segment_sum(feat, idxp):
    zf = jnp.zeros((ROWS_PER_TILE, D), jnp.float32)
    ones = jnp.ones((CHUNK, D), jnp.float32)

    mesh = plsc.VectorSubcoreMesh(core_axis_name="c", subcore_axis_name="s",
                                  num_cores=NC, num_subcores=NS)

    @functools.partial(
        pl.kernel,
        out_type=(
            jax.ShapeDtypeStruct((NC, NP, D), jnp.float32),
            jax.ShapeDtypeStruct((NC, NP, D), jnp.float32),
        ),
        mesh=mesh,
        scratch_types=[
            pltpu.VMEM((2, CHUNK), jnp.int32),
            pltpu.VMEM((CHUNK, D), jnp.float32),
            pltpu.VMEM((CHUNK, D), jnp.float32),
            pltpu.VMEM_SHARED((NP, D), jnp.float32),
            pltpu.SemaphoreType.DMA,
        ],
    )
    def sc_kernel(feat_hbm, idx_hbm, zf_hbm, ones_hbm,
                  sum_hbm, deg_hbm,
                  sd_v, rows_v, ones_v, acc_s, sem_a):
        c = lax.axis_index("c")
        s = lax.axis_index("s")
        wid = s * NC + c
        base = s * ROWS_PER_TILE

        pltpu.sync_copy(zf_hbm, acc_s.at[pl.ds(base, ROWS_PER_TILE)])
        pltpu.sync_copy(ones_hbm, ones_v)
        plsc.subcore_barrier()

        def feat_body(j, carry):
            pltpu.sync_copy(idx_hbm.at[wid, j], sd_v)
            pltpu.async_copy(feat_hbm.at[sd_v.at[0]], rows_v, sem_a).wait()
            pltpu.sync_copy(rows_v, acc_s.at[sd_v.at[1]], add=True)
            return carry

        lax.fori_loop(0, NCH, feat_body, 0)
        plsc.subcore_barrier()

        pltpu.sync_copy(acc_s.at[pl.ds(base, ROWS_PER_TILE)],
                        sum_hbm.at[c, pl.ds(base, ROWS_PER_TILE), :])
        pltpu.sync_copy(zf_hbm, acc_s.at[pl.ds(base, ROWS_PER_TILE)])
        plsc.subcore_barrier()

        def deg_body(j, carry):
            pltpu.sync_copy(idx_hbm.at[wid, j], sd_v)
            pltpu.sync_copy(ones_v, acc_s.at[sd_v.at[1]], add=True)
            return carry

        lax.fori_loop(0, NCH, deg_body, 0)
        plsc.subcore_barrier()

        pltpu.sync_copy(acc_s.at[pl.ds(base, ROWS_PER_TILE)],
                        deg_hbm.at[c, pl.ds(base, ROWS_PER_TILE), :])

    return sc_kernel(feat, idxp, zf, ones)


def _tc_body(sum_ref, deg_ref, feat_ref, wn_ref, ws_ref, b_ref, out_ref):
    s = sum_ref[0] + sum_ref[1]
    d = deg_ref[0, :, 0:1] + deg_ref[1, :, 0:1]
    mean = s / jnp.maximum(d, 1.0)
    h = jnp.dot(mean, wn_ref[...], preferred_element_type=jnp.float32)
    h += jnp.dot(feat_ref[...], ws_ref[...], preferred_element_type=jnp.float32)
    out_ref[...] = h + b_ref[...]


def _tc_combine(sums, degs, featp, W_neigh, W_self, b_self):
    blk = 1024
    grid = NP // blk
    return pl.pallas_call(
        _tc_body,
        grid=(grid,),
        in_specs=[
            pl.BlockSpec((NC, blk, D), lambda i: (0, i, 0)),
            pl.BlockSpec((NC, blk, D), lambda i: (0, i, 0)),
            pl.BlockSpec((blk, D), lambda i: (i, 0)),
            pl.BlockSpec((D, D), lambda i: (0, 0)),
            pl.BlockSpec((D, D), lambda i: (0, 0)),
            pl.BlockSpec((1, D), lambda i: (0, 0)),
        ],
        out_specs=pl.BlockSpec((blk, D), lambda i: (i, 0)),
        out_shape=jax.ShapeDtypeStruct((NP, D), jnp.float32),
    )(sums, degs, featp, W_neigh, W_self, b_self)


@jax.jit
def kernel(feat, edge_index, W_neigh, W_self, b_self):
    src = edge_index[0].astype(jnp.int32)
    dst = edge_index[1].astype(jnp.int32)
    pad = EP - E
    srcp = jnp.concatenate([src, jnp.zeros((pad,), jnp.int32)]).reshape(
        NW, NCH, CHUNK)
    dummy = N + (jnp.arange(pad, dtype=jnp.int32) % (NP - N))
    dstp = jnp.concatenate([dst, dummy]).reshape(NW, NCH, CHUNK)
    idxp = jnp.stack([srcp, dstp], axis=2)

    sums, degs = _sc_segment_sum(feat, idxp)

    featp = jnp.concatenate(
        [feat, jnp.zeros((NP - N, D), jnp.float32)], axis=0)
    out = _tc_combine(sums, degs, featp, W_neigh, W_self,
                      b_self.reshape(1, D))
    return out[:N]
